# baseline (speedup 1.0000x reference)
import functools

import jax
import jax.numpy as jnp
from jax import lax
from jax.experimental import pallas as pl
from jax.experimental.pallas import tpu as pltpu

_LN_EPS = 1e-5


def _fused_kernel(x_ref, wt_ref, b_ref, g_ref, beta_ref, o_ref):
    x = x_ref[...]
    y = jnp.dot(x.astype(jnp.bfloat16), wt_ref[...],
                preferred_element_type=jnp.float32)
    z = y + x + b_ref[...]
    d = z.shape[-1]
    inv_d = jnp.float32(1.0 / d)
    mean = jnp.sum(z, axis=-1, keepdims=True) * inv_d
    ex2 = jnp.sum(z * z, axis=-1, keepdims=True) * inv_d
    var = jnp.maximum(ex2 - mean * mean, 0.0)
    rstd = lax.rsqrt(var + _LN_EPS)
    scale = rstd * g_ref[...]
    shift = beta_ref[...] - mean * scale
    o_ref[...] = (z * scale + shift).astype(o_ref.dtype)


@functools.partial(jax.jit, static_argnames=("tm",))
def _forward(x, w, b, gamma, beta, *, tm=512):
    B, S, D = x.shape
    R = B * S
    TM = min(tm, R)
    n_row = pl.cdiv(R, TM)
    R_pad = n_row * TM

    x2 = x.reshape(R, D)
    if R_pad != R:
        x2 = jnp.pad(x2, ((0, R_pad - R), (0, 0)))
    wt = jnp.asarray(w).T.astype(jnp.bfloat16)
    b2 = b.reshape(1, D).astype(jnp.float32)
    g2 = gamma.reshape(1, D).astype(jnp.float32)
    beta2 = beta.reshape(1, D).astype(jnp.float32)

    out2 = pl.pallas_call(
        _fused_kernel,
        out_shape=jax.ShapeDtypeStruct((R_pad, D), x.dtype),
        grid=(n_row,),
        in_specs=[
            pl.BlockSpec((TM, D), lambda i: (i, 0)),
            pl.BlockSpec((D, D), lambda i: (0, 0)),
            pl.BlockSpec((1, D), lambda i: (0, 0)),
            pl.BlockSpec((1, D), lambda i: (0, 0)),
            pl.BlockSpec((1, D), lambda i: (0, 0)),
        ],
        out_specs=pl.BlockSpec((TM, D), lambda i: (i, 0)),
        compiler_params=pltpu.CompilerParams(
            dimension_semantics=("arbitrary",),
            vmem_limit_bytes=56 * 1024 * 1024,
        ),
    )(x2, wt, b2, g2, beta2)
    return out2[:R].reshape(B, S, D)


def kernel(x, w, b, gamma, beta):
    return _forward(x, w, b, gamma, beta, tm=2048)


# speedup vs baseline: 1.0457x; 1.0457x over previous
import functools

import jax
import jax.numpy as jnp
from jax import lax
from jax.experimental import pallas as pl
from jax.experimental.pallas import tpu as pltpu

_LN_EPS = 1e-5


def _ln_block(z, g, beta, out_dtype):
    d = z.shape[-1]
    inv_d = jnp.float32(1.0 / d)
    mean = jnp.sum(z, axis=-1, keepdims=True) * inv_d
    ex2 = jnp.sum(z * z, axis=-1, keepdims=True) * inv_d
    var = jnp.maximum(ex2 - mean * mean, 0.0)
    rstd = lax.rsqrt(var + _LN_EPS)
    scale = rstd * g
    shift = beta - mean * scale
    return (z * scale + shift).astype(out_dtype)


def _pipeline_kernel(x_hbm, wt_ref, b_ref, g_ref, beta_ref, o_hbm,
                     x_buf, o_buf, in_sems, out_sems,
                     *, tm, n_steps, n_in, n_out):
    def dma_in(slot, step):
        pltpu.make_async_copy(
            x_hbm.at[pl.ds(step * tm, tm), :], x_buf.at[slot],
            in_sems.at[slot]).start()

    def wait_in(slot):
        pltpu.make_async_copy(
            x_buf.at[slot], x_buf.at[slot], in_sems.at[slot]).wait()

    def dma_out(slot, step):
        pltpu.make_async_copy(
            o_buf.at[slot], o_hbm.at[pl.ds(step * tm, tm), :],
            out_sems.at[slot]).start()

    def wait_out(slot):
        pltpu.make_async_copy(
            o_buf.at[slot], o_buf.at[slot], out_sems.at[slot]).wait()

    for i in range(min(n_in, n_steps)):
        dma_in(i, i)

    b = b_ref[...]
    g = g_ref[...]
    beta = beta_ref[...]

    def body(step, carry):
        cur = lax.rem(step, n_in)
        ocur = lax.rem(step, n_out)
        wait_in(cur)

        @pl.when(step >= n_out)
        def _():
            wait_out(ocur)

        x = x_buf[cur]
        y = jnp.dot(x.astype(jnp.bfloat16), wt_ref[...],
                    preferred_element_type=jnp.float32)
        z = y + x + b
        o_buf[ocur] = _ln_block(z, g, beta, o_buf.dtype)
        dma_out(ocur, step)

        @pl.when(step + n_in < n_steps)
        def _():
            dma_in(cur, step + n_in)

        return carry

    lax.fori_loop(0, n_steps, body, 0)
    for i in range(min(n_out, n_steps)):
        wait_out(i)


@functools.partial(jax.jit, static_argnames=("tm", "n_in", "n_out"))
def _forward(x, w, b, gamma, beta, *, tm=512, n_in=4, n_out=4):
    B, S, D = x.shape
    R = B * S
    TM = min(tm, R)
    n_steps = pl.cdiv(R, TM)
    R_pad = n_steps * TM

    x2 = x.reshape(R, D)
    if R_pad != R:
        x2 = jnp.pad(x2, ((0, R_pad - R), (0, 0)))
    wt = jnp.asarray(w).T.astype(jnp.bfloat16)
    b2 = b.reshape(1, D).astype(jnp.float32)
    g2 = gamma.reshape(1, D).astype(jnp.float32)
    beta2 = beta.reshape(1, D).astype(jnp.float32)

    kernel_fn = functools.partial(
        _pipeline_kernel, tm=TM, n_steps=n_steps, n_in=n_in, n_out=n_out)
    out2 = pl.pallas_call(
        kernel_fn,
        out_shape=jax.ShapeDtypeStruct((R_pad, D), x.dtype),
        in_specs=[
            pl.BlockSpec(memory_space=pl.ANY),
            pl.BlockSpec(memory_space=pltpu.VMEM),
            pl.BlockSpec(memory_space=pltpu.VMEM),
            pl.BlockSpec(memory_space=pltpu.VMEM),
            pl.BlockSpec(memory_space=pltpu.VMEM),
        ],
        out_specs=pl.BlockSpec(memory_space=pl.ANY),
        scratch_shapes=[
            pltpu.VMEM((n_in, TM, D), x.dtype),
            pltpu.VMEM((n_out, TM, D), x.dtype),
            pltpu.SemaphoreType.DMA((n_in,)),
            pltpu.SemaphoreType.DMA((n_out,)),
        ],
        compiler_params=pltpu.CompilerParams(
            vmem_limit_bytes=56 * 1024 * 1024,
        ),
    )(x2, wt, b2, g2, beta2)
    return out2[:R].reshape(B, S, D)


def kernel(x, w, b, gamma, beta):
    return _forward(x, w, b, gamma, beta, tm=512, n_in=4, n_out=4)
